# baseline (device time: 1113549 ns/iter reference)
import jax
import jax.numpy as jnp
from jax import lax
from jax.experimental import pallas as pl
from jax.experimental.pallas import tpu as pltpu

N_DEV = 32
B = 2048
D = 1024
CH = B // N_DEV


def _neighbor_barrier(left, right):
    barrier = pltpu.get_barrier_semaphore()
    for nbr in (left, right):
        pl.semaphore_signal(
            barrier, inc=1, device_id=(nbr,),
            device_id_type=pl.DeviceIdType.MESH,
        )
    pl.semaphore_wait(barrier, 2)


def _ag_body(x_ref, out_ref, comm_ref, send_sems, recv_sems):
    my = lax.axis_index("i")
    left = (my - 1) % N_DEV
    right = (my + 1) % N_DEV
    _neighbor_barrier(left, right)

    out_ref[pl.ds(my * CH, CH), :] = x_ref[...]
    comm_ref[0, :, :] = x_ref[...]

    for h in range(N_DEV - 1):
        ss, rr = h % 2, (h + 1) % 2
        rdma = pltpu.make_async_remote_copy(
            src_ref=comm_ref.at[ss],
            dst_ref=comm_ref.at[rr],
            send_sem=send_sems.at[h],
            recv_sem=recv_sems.at[h],
            device_id=(right,),
            device_id_type=pl.DeviceIdType.MESH,
        )
        rdma.start()
        rdma.wait()
        origin = (my - h - 1) % N_DEV
        out_ref[pl.ds(origin * CH, CH), :] = comm_ref[rr]


def _allgather(x):
    return pl.pallas_call(
        _ag_body,
        out_shape=jax.ShapeDtypeStruct((B, D), x.dtype),
        in_specs=[pl.BlockSpec(memory_space=pltpu.VMEM)],
        out_specs=pl.BlockSpec(memory_space=pltpu.VMEM),
        scratch_shapes=[
            pltpu.VMEM((2, CH, D), x.dtype),
            pltpu.SemaphoreType.DMA((N_DEV - 1,)),
            pltpu.SemaphoreType.DMA((N_DEV - 1,)),
        ],
        compiler_params=pltpu.CompilerParams(collective_id=0),
    )(x)


_RB = 256


def _layer_body(x_ref, win_ref, wout_ref, out_ref):
    h = jnp.maximum(
        jnp.dot(x_ref[...], win_ref[...], preferred_element_type=jnp.float32),
        0.0,
    )
    out_ref[...] = jnp.dot(h, wout_ref[...], preferred_element_type=jnp.float32)


def _layer(x_full, win, wout):
    hdim = win.shape[1]
    return pl.pallas_call(
        _layer_body,
        grid=(B // _RB,),
        in_specs=[
            pl.BlockSpec((_RB, D), lambda r: (r, 0)),
            pl.BlockSpec((D, hdim), lambda r: (0, 0)),
            pl.BlockSpec((hdim, D), lambda r: (0, 0)),
        ],
        out_specs=pl.BlockSpec((_RB, D), lambda r: (r, 0)),
        out_shape=jax.ShapeDtypeStruct((B, D), jnp.float32),
    )(x_full, win, wout)


def _ar_body(p_ref, out_ref, comm_ref, rs_send, rs_recv, ag_send, ag_recv):
    my = lax.axis_index("i")
    left = (my - 1) % N_DEV
    right = (my + 1) % N_DEV
    _neighbor_barrier(left, right)

    comm_ref[0, :, :] = p_ref[pl.ds(my * CH, CH), :]
    for h in range(N_DEV - 1):
        ss, rr = h % 2, (h + 1) % 2
        rdma = pltpu.make_async_remote_copy(
            src_ref=comm_ref.at[ss],
            dst_ref=comm_ref.at[rr],
            send_sem=rs_send.at[h],
            recv_sem=rs_recv.at[h],
            device_id=(right,),
            device_id_type=pl.DeviceIdType.MESH,
        )
        rdma.start()
        rdma.wait()
        c = (my - h - 1) % N_DEV
        comm_ref[rr, :, :] = comm_ref[rr] + p_ref[pl.ds(c * CH, CH), :]

    own = (my + 1) % N_DEV
    out_ref[pl.ds(own * CH, CH), :] = comm_ref[1]

    for h in range(N_DEV - 1):
        ss, rr = (h + 1) % 2, h % 2
        rdma = pltpu.make_async_remote_copy(
            src_ref=comm_ref.at[ss],
            dst_ref=comm_ref.at[rr],
            send_sem=ag_send.at[h],
            recv_sem=ag_recv.at[h],
            device_id=(right,),
            device_id_type=pl.DeviceIdType.MESH,
        )
        rdma.start()
        rdma.wait()
        origin = (my - h) % N_DEV
        out_ref[pl.ds(origin * CH, CH), :] = comm_ref[rr]


def _allreduce(p):
    return pl.pallas_call(
        _ar_body,
        out_shape=jax.ShapeDtypeStruct((B, D), p.dtype),
        in_specs=[pl.BlockSpec(memory_space=pltpu.VMEM)],
        out_specs=pl.BlockSpec(memory_space=pltpu.VMEM),
        scratch_shapes=[
            pltpu.VMEM((2, CH, D), p.dtype),
            pltpu.SemaphoreType.DMA((N_DEV - 1,)),
            pltpu.SemaphoreType.DMA((N_DEV - 1,)),
            pltpu.SemaphoreType.DMA((N_DEV - 1,)),
            pltpu.SemaphoreType.DMA((N_DEV - 1,)),
        ],
        compiler_params=pltpu.CompilerParams(collective_id=1),
    )(p)


def kernel(x, Win0, Wout0, Win1, Wout1, Win2, Wout2):
    x_full = _allgather(x)
    for win, wout in ((Win0, Wout0), (Win1, Wout1), (Win2, Wout2)):
        partial = _layer(x_full, win, wout)
        x_full = _allreduce(partial)
    return x_full
